# baseline (speedup 1.0000x reference)
import functools

import jax
import jax.numpy as jnp
from jax import lax
from jax.experimental import pallas as pl
from jax.experimental.pallas import tpu as pltpu
from jax.experimental.pallas import tpu_sc as plsc

_NC = 2
_NS = 16
_NW = _NC * _NS
_CHUNK = 128


def _sc_mesh():
    return plsc.VectorSubcoreMesh(core_axis_name="c", subcore_axis_name="s")


def _gather_body(bpw, u_tbl, i_tbl, c_tbl, co_tbl, p_tbl,
                 uid, iid, cid, coid, pid,
                 u_out, i_out, c_out, co_out, p_out,
                 uix, iix, cix, coix, pix,
                 buf0, buf1, sem0, sem1, osem0, osem1):
    idx_rows = bpw // _CHUNK
    wid = lax.axis_index("s") * _NC + lax.axis_index("c")
    base = wid * bpw
    irow = wid * idx_rows
    fields = [(uid, uix, u_tbl, u_out),
              (iid, iix, i_tbl, i_out),
              (cid, cix, c_tbl, c_out),
              (coid, coix, co_tbl, co_out),
              (pid, pix, p_tbl, p_out)]
    for ids, ixv, _, _ in fields:
        pltpu.sync_copy(ids.at[pl.ds(irow, idx_rows)], ixv)

    bufs = (buf0, buf1)
    gsems = (sem0, sem1)
    osems = (osem0, osem1)
    chunks = []
    for _, ixv, tbl, outr in fields:
        for j in range(idx_rows):
            chunks.append((ixv, j, tbl, outr, base + j * _CHUNK))
    n = len(chunks)
    gcps = [None, None]
    scps = [None, None]

    def fire(k):
        ixv, j, tbl, outr, _ = chunks[k]
        b = k % 2
        gcps[b] = pltpu.async_copy(tbl.at[ixv.at[j]], bufs[b], gsems[b])

    fire(0)
    for k in range(n):
        b = k % 2
        ixv, j, tbl, outr, off = chunks[k]
        if k + 1 < n:
            nb = (k + 1) % 2
            if scps[nb] is not None:
                scps[nb].wait()
                scps[nb] = None
            fire(k + 1)
        gcps[b].wait()
        scps[b] = pltpu.async_copy(bufs[b], outr.at[pl.ds(off, _CHUNK)],
                                   osems[b])
    for b in range(2):
        if scps[b] is not None:
            scps[b].wait()


@functools.lru_cache(maxsize=None)
def _make_gather(batch):
    bpw = batch // _NW
    idx_rows = bpw // _CHUNK
    f32 = jnp.float32
    out = jax.ShapeDtypeStruct((batch, 128), f32)
    return pl.kernel(
        functools.partial(_gather_body, bpw),
        out_type=[out] * 5,
        mesh=_sc_mesh(),
        scratch_types=[pltpu.VMEM((idx_rows, _CHUNK), jnp.int32)] * 5
                      + [pltpu.VMEM((_CHUNK, 128), f32)] * 2
                      + [pltpu.SemaphoreType.DMA] * 4,
        compiler_params=pltpu.CompilerParams(use_tc_tiling_on_sc=True),
    )


def _mlp_body(upk, ipk, gc, gco, gp, sf,
              w1u, w1i, w1s, b1, w2, b2, w3, b3, w4, b4, out):
    f32 = jnp.float32
    hi = jax.lax.Precision.HIGHEST
    pu = sf[:, 2:3]
    pi = sf[:, 3:4]
    ue = jnp.where(pu > 0.5, upk[:, 64:128], upk[:, 0:64])
    ie = jnp.where(pi > 0.5, ipk[:, 64:128], ipk[:, 0:64])
    h = jnp.dot(ue, w1u[...], precision=hi, preferred_element_type=f32)
    h = h + jnp.dot(ie, w1i[...], precision=hi, preferred_element_type=f32)
    h = h + jnp.dot(sf[...], w1s[...], precision=hi, preferred_element_type=f32)
    h = h + gc[...] + gco[...] + gp[...] + b1[...]
    h = jnp.maximum(h, 0.0)
    h = jnp.maximum(jnp.dot(h, w2[...], precision=hi, preferred_element_type=f32) + b2[...], 0.0)
    h = jnp.maximum(jnp.dot(h, w3[...], precision=hi, preferred_element_type=f32) + b3[...], 0.0)
    o = jnp.dot(h, w4[...], precision=hi, preferred_element_type=f32) + b4[...]
    out[...] = jax.nn.sigmoid(o)


def _mlp(upk, ipk, gc, gco, gp, sf, w1u, w1i, w1s, b1,
         w2, b2, w3, b3, w4, b4, tile=1024, interpret=False):
    batch = upk.shape[0]

    def row(d):
        return pl.BlockSpec((tile, d), lambda i: (i, 0))

    def const(a):
        return pl.BlockSpec(a.shape, lambda i: (0,) * a.ndim)

    return pl.pallas_call(
        _mlp_body,
        grid=(batch // tile,),
        in_specs=[row(128), row(128), row(128), row(128), row(128),
                  row(sf.shape[1]),
                  const(w1u), const(w1i), const(w1s), const(b1),
                  const(w2), const(b2), const(w3), const(b3),
                  const(w4), const(b4)],
        out_specs=pl.BlockSpec((tile, 1), lambda i: (i, 0)),
        out_shape=jax.ShapeDtypeStruct((batch, 1), jnp.float32),
        interpret=interpret,
    )(upk, ipk, gc, gco, gp, sf, w1u, w1i, w1s, b1, w2, b2, w3, b3, w4, b4)


def kernel(user_id, item_id, category, color, price_range, num_colors,
           price_numeric, user_table, item_table, category_table, color_table,
           price_range_table, W1, b1, W2, b2, W3, b3, W4, b4):
    batch = user_id.shape[0]
    du = user_table.shape[1]
    ds = category_table.shape[1]
    hi = jax.lax.Precision.HIGHEST

    ut = user_table.reshape(-1, 128)
    it = item_table.reshape(-1, 128)
    o1, o2, o3, o4, o5 = du, 2 * du, 2 * du + ds, 2 * du + 2 * ds, 2 * du + 3 * ds
    gc_tbl = jnp.dot(category_table, W1[o2:o3], precision=hi)
    gco_tbl = jnp.dot(color_table, W1[o3:o4], precision=hi)
    gp_tbl = jnp.dot(price_range_table, W1[o4:o5], precision=hi)

    gather = _make_gather(batch)
    upk, ipk, gc, gco, gp = gather(
        ut, it, gc_tbl, gco_tbl, gp_tbl,
        (user_id >> 1).reshape(-1, _CHUNK), (item_id >> 1).reshape(-1, _CHUNK),
        category.reshape(-1, _CHUNK), color.reshape(-1, _CHUNK),
        price_range.reshape(-1, _CHUNK))

    sf = jnp.stack([num_colors, price_numeric,
                    (user_id & 1).astype(jnp.float32),
                    (item_id & 1).astype(jnp.float32)], axis=1)
    w1s = jnp.concatenate([W1[o5:], jnp.zeros((2, W1.shape[1]), jnp.float32)],
                         axis=0)
    out = _mlp(upk, ipk, gc, gco, gp, sf,
               W1[:o1], W1[o1:o2], w1s,
               b1.reshape(1, -1), W2, b2.reshape(1, -1), W3, b3.reshape(1, -1),
               W4, b4.reshape(1, 1))
    return out[:, 0]


# speedup vs baseline: 1.0014x; 1.0014x over previous
import functools

import jax
import jax.numpy as jnp
from jax import lax
from jax.experimental import pallas as pl
from jax.experimental.pallas import tpu as pltpu
from jax.experimental.pallas import tpu_sc as plsc

_NC = 2
_NS = 16
_NW = _NC * _NS
_CHUNK = 128


def _sc_mesh():
    return plsc.VectorSubcoreMesh(core_axis_name="c", subcore_axis_name="s")


def _gather_body(bpw, u_tbl, i_tbl, c_tbl, co_tbl, p_tbl,
                 uid, iid, cid, coid, pid,
                 u_out, i_out, c_out, co_out, p_out,
                 uix, iix, cix, coix, pix,
                 buf0, buf1, sem0, sem1, osem0, osem1):
    idx_rows = bpw // _CHUNK
    wid = lax.axis_index("s") * _NC + lax.axis_index("c")
    base = wid * bpw
    irow = wid * idx_rows
    fields = [(uid, uix, u_tbl, u_out),
              (iid, iix, i_tbl, i_out),
              (cid, cix, c_tbl, c_out),
              (coid, coix, co_tbl, co_out),
              (pid, pix, p_tbl, p_out)]
    for ids, ixv, _, _ in fields:
        pltpu.sync_copy(ids.at[pl.ds(irow, idx_rows)], ixv)

    bufs = (buf0, buf1)
    gsems = (sem0, sem1)
    osems = (osem0, osem1)
    chunks = []
    for _, ixv, tbl, outr in fields:
        for j in range(idx_rows):
            chunks.append((ixv, j, tbl, outr, base + j * _CHUNK))
    n = len(chunks)
    gcps = [None, None]
    scps = [None, None]

    def fire(k):
        ixv, j, tbl, outr, _ = chunks[k]
        b = k % 2
        gcps[b] = pltpu.async_copy(tbl.at[ixv.at[j]], bufs[b], gsems[b])

    fire(0)
    for k in range(n):
        b = k % 2
        ixv, j, tbl, outr, off = chunks[k]
        if k + 1 < n:
            nb = (k + 1) % 2
            if scps[nb] is not None:
                scps[nb].wait()
                scps[nb] = None
            fire(k + 1)
        gcps[b].wait()
        scps[b] = pltpu.async_copy(bufs[b], outr.at[pl.ds(off, _CHUNK)],
                                   osems[b])
    for b in range(2):
        if scps[b] is not None:
            scps[b].wait()


@functools.lru_cache(maxsize=None)
def _make_gather(batch):
    bpw = batch // _NW
    idx_rows = bpw // _CHUNK
    f32 = jnp.float32
    out = jax.ShapeDtypeStruct((batch, 128), f32)
    return pl.kernel(
        functools.partial(_gather_body, bpw),
        out_type=[out] * 5,
        mesh=_sc_mesh(),
        scratch_types=[pltpu.VMEM((idx_rows, _CHUNK), jnp.int32)] * 5
                      + [pltpu.VMEM((_CHUNK, 128), f32)] * 2
                      + [pltpu.SemaphoreType.DMA] * 4,
        compiler_params=pltpu.CompilerParams(use_tc_tiling_on_sc=True, needs_layout_passes=False),
    )


def _mlp_body(upk, ipk, gc, gco, gp, sf,
              w1u, w1i, w1s, b1, w2, b2, w3, b3, w4, b4, out):
    f32 = jnp.float32
    hi = jax.lax.Precision.HIGHEST
    pu = sf[:, 2:3]
    pi = sf[:, 3:4]
    ue = jnp.where(pu > 0.5, upk[:, 64:128], upk[:, 0:64])
    ie = jnp.where(pi > 0.5, ipk[:, 64:128], ipk[:, 0:64])
    h = jnp.dot(ue, w1u[...], precision=hi, preferred_element_type=f32)
    h = h + jnp.dot(ie, w1i[...], precision=hi, preferred_element_type=f32)
    h = h + jnp.dot(sf[...], w1s[...], precision=hi, preferred_element_type=f32)
    h = h + gc[...] + gco[...] + gp[...] + b1[...]
    h = jnp.maximum(h, 0.0)
    h = jnp.maximum(jnp.dot(h, w2[...], precision=hi, preferred_element_type=f32) + b2[...], 0.0)
    h = jnp.maximum(jnp.dot(h, w3[...], precision=hi, preferred_element_type=f32) + b3[...], 0.0)
    o = jnp.dot(h, w4[...], precision=hi, preferred_element_type=f32) + b4[...]
    out[...] = jax.nn.sigmoid(o)


def _mlp(upk, ipk, gc, gco, gp, sf, w1u, w1i, w1s, b1,
         w2, b2, w3, b3, w4, b4, tile=1024, interpret=False):
    batch = upk.shape[0]

    def row(d):
        return pl.BlockSpec((tile, d), lambda i: (i, 0))

    def const(a):
        return pl.BlockSpec(a.shape, lambda i: (0,) * a.ndim)

    return pl.pallas_call(
        _mlp_body,
        grid=(batch // tile,),
        in_specs=[row(128), row(128), row(128), row(128), row(128),
                  row(sf.shape[1]),
                  const(w1u), const(w1i), const(w1s), const(b1),
                  const(w2), const(b2), const(w3), const(b3),
                  const(w4), const(b4)],
        out_specs=pl.BlockSpec((tile, 1), lambda i: (i, 0)),
        out_shape=jax.ShapeDtypeStruct((batch, 1), jnp.float32),
        interpret=interpret,
    )(upk, ipk, gc, gco, gp, sf, w1u, w1i, w1s, b1, w2, b2, w3, b3, w4, b4)


def kernel(user_id, item_id, category, color, price_range, num_colors,
           price_numeric, user_table, item_table, category_table, color_table,
           price_range_table, W1, b1, W2, b2, W3, b3, W4, b4):
    batch = user_id.shape[0]
    du = user_table.shape[1]
    ds = category_table.shape[1]
    hi = jax.lax.Precision.HIGHEST

    ut = user_table.reshape(-1, 128)
    it = item_table.reshape(-1, 128)
    o1, o2, o3, o4, o5 = du, 2 * du, 2 * du + ds, 2 * du + 2 * ds, 2 * du + 3 * ds
    gc_tbl = jnp.dot(category_table, W1[o2:o3], precision=hi)
    gco_tbl = jnp.dot(color_table, W1[o3:o4], precision=hi)
    gp_tbl = jnp.dot(price_range_table, W1[o4:o5], precision=hi)

    gather = _make_gather(batch)
    upk, ipk, gc, gco, gp = gather(
        ut, it, gc_tbl, gco_tbl, gp_tbl,
        (user_id >> 1).reshape(-1, _CHUNK), (item_id >> 1).reshape(-1, _CHUNK),
        category.reshape(-1, _CHUNK), color.reshape(-1, _CHUNK),
        price_range.reshape(-1, _CHUNK))

    sf = jnp.stack([num_colors, price_numeric,
                    (user_id & 1).astype(jnp.float32),
                    (item_id & 1).astype(jnp.float32)], axis=1)
    w1s = jnp.concatenate([W1[o5:], jnp.zeros((2, W1.shape[1]), jnp.float32)],
                         axis=0)
    out = _mlp(upk, ipk, gc, gco, gp, sf,
               W1[:o1], W1[o1:o2], w1s,
               b1.reshape(1, -1), W2, b2.reshape(1, -1), W3, b3.reshape(1, -1),
               W4, b4.reshape(1, 1))
    return out[:, 0]


# speedup vs baseline: 1.5372x; 1.5350x over previous
import functools

import jax
import jax.numpy as jnp
from jax import lax
from jax.experimental import pallas as pl
from jax.experimental.pallas import tpu as pltpu
from jax.experimental.pallas import tpu_sc as plsc

_NC = 2
_NS = 16
_NW = _NC * _NS
_CHUNK = 128


def _sc_mesh():
    return plsc.VectorSubcoreMesh(core_axis_name="c", subcore_axis_name="s")


def _gather_body(bpw, u_tbl, i_tbl, c_tbl, co_tbl, p_tbl,
                 uid, iid, cid, coid, pid,
                 u_out, i_out, c_out, co_out, p_out,
                 uix, iix, cix, coix, pix,
                 buf0, buf1, sem0, sem1, osem0, osem1):
    idx_rows = bpw // _CHUNK
    wid = lax.axis_index("s") * _NC + lax.axis_index("c")
    base = wid * bpw
    irow = wid * idx_rows
    fields = [(uid, uix, u_tbl, u_out),
              (iid, iix, i_tbl, i_out),
              (cid, cix, c_tbl, c_out),
              (coid, coix, co_tbl, co_out),
              (pid, pix, p_tbl, p_out)]
    for ids, ixv, _, _ in fields:
        pltpu.sync_copy(ids.at[pl.ds(irow, idx_rows)], ixv)

    bufs = (buf0, buf1)
    gsems = (sem0, sem1)
    osems = (osem0, osem1)
    chunks = []
    for _, ixv, tbl, outr in fields:
        for j in range(idx_rows):
            chunks.append((ixv, j, tbl, outr, base + j * _CHUNK))
    n = len(chunks)
    gcps = [None, None]
    scps = [None, None]

    def fire(k):
        ixv, j, tbl, outr, _ = chunks[k]
        b = k % 2
        gcps[b] = pltpu.async_copy(tbl.at[ixv.at[j]], bufs[b], gsems[b])

    fire(0)
    for k in range(n):
        b = k % 2
        ixv, j, tbl, outr, off = chunks[k]
        if k + 1 < n:
            nb = (k + 1) % 2
            if scps[nb] is not None:
                scps[nb].wait()
                scps[nb] = None
            fire(k + 1)
        gcps[b].wait()
        scps[b] = pltpu.async_copy(bufs[b], outr.at[pl.ds(off, _CHUNK)],
                                   osems[b])
    for b in range(2):
        if scps[b] is not None:
            scps[b].wait()


@functools.lru_cache(maxsize=None)
def _make_gather(batch):
    bpw = batch // _NW
    idx_rows = bpw // _CHUNK
    f32 = jnp.float32
    out = jax.ShapeDtypeStruct((batch, 128), f32)
    return pl.kernel(
        functools.partial(_gather_body, bpw),
        out_type=[out] * 5,
        mesh=_sc_mesh(),
        scratch_types=[pltpu.VMEM((idx_rows, _CHUNK), jnp.int32)] * 5
                      + [pltpu.VMEM((_CHUNK, 128), f32)] * 2
                      + [pltpu.SemaphoreType.DMA] * 4,
        compiler_params=pltpu.CompilerParams(use_tc_tiling_on_sc=True, needs_layout_passes=False),
    )


def _tpack_body(xa, xb, out):
    ya = jnp.transpose(xa[...])
    yb = jnp.transpose(xb[...])
    out[...] = jnp.concatenate([ya, yb], axis=1)


def _transpose_pack(table_t, tile=2048):
    n = table_t.shape[1]
    grid = -(-(n // 2) // tile)
    h = grid * tile
    last = n // tile
    packed = pl.pallas_call(
        _tpack_body,
        grid=(grid,),
        in_specs=[pl.BlockSpec((64, tile), lambda i: (0, i)),
                  pl.BlockSpec((64, tile),
                               lambda i: (0, jnp.minimum(i + grid, last)))],
        out_specs=pl.BlockSpec((tile, 128), lambda i: (i, 0)),
        out_shape=jax.ShapeDtypeStruct((h, 128), jnp.float32),
    )(table_t, table_t)
    return packed, h


def _mlp_body(upk, ipk, gc, gco, gp, sf,
              w1u, w1i, w1s, b1, w2, b2, w3, b3, w4, b4, out):
    f32 = jnp.float32
    hi = jax.lax.Precision.HIGHEST
    pu = sf[:, 2:3]
    pi = sf[:, 3:4]
    ue = jnp.where(pu > 0.5, upk[:, 64:128], upk[:, 0:64])
    ie = jnp.where(pi > 0.5, ipk[:, 64:128], ipk[:, 0:64])
    h = jnp.dot(ue, w1u[...], precision=hi, preferred_element_type=f32)
    h = h + jnp.dot(ie, w1i[...], precision=hi, preferred_element_type=f32)
    h = h + jnp.dot(sf[...], w1s[...], precision=hi, preferred_element_type=f32)
    h = h + gc[...] + gco[...] + gp[...] + b1[...]
    h = jnp.maximum(h, 0.0)
    h = jnp.maximum(jnp.dot(h, w2[...], precision=hi, preferred_element_type=f32) + b2[...], 0.0)
    h = jnp.maximum(jnp.dot(h, w3[...], precision=hi, preferred_element_type=f32) + b3[...], 0.0)
    o = jnp.dot(h, w4[...], precision=hi, preferred_element_type=f32) + b4[...]
    out[...] = jax.nn.sigmoid(o)


def _mlp(upk, ipk, gc, gco, gp, sf, w1u, w1i, w1s, b1,
         w2, b2, w3, b3, w4, b4, tile=1024, interpret=False):
    batch = upk.shape[0]

    def row(d):
        return pl.BlockSpec((tile, d), lambda i: (i, 0))

    def const(a):
        return pl.BlockSpec(a.shape, lambda i: (0,) * a.ndim)

    return pl.pallas_call(
        _mlp_body,
        grid=(batch // tile,),
        in_specs=[row(128), row(128), row(128), row(128), row(128),
                  row(sf.shape[1]),
                  const(w1u), const(w1i), const(w1s), const(b1),
                  const(w2), const(b2), const(w3), const(b3),
                  const(w4), const(b4)],
        out_specs=pl.BlockSpec((tile, 1), lambda i: (i, 0)),
        out_shape=jax.ShapeDtypeStruct((batch, 1), jnp.float32),
        interpret=interpret,
    )(upk, ipk, gc, gco, gp, sf, w1u, w1i, w1s, b1, w2, b2, w3, b3, w4, b4)


def kernel(user_id, item_id, category, color, price_range, num_colors,
           price_numeric, user_table, item_table, category_table, color_table,
           price_range_table, W1, b1, W2, b2, W3, b3, W4, b4):
    batch = user_id.shape[0]
    du = user_table.shape[1]
    ds = category_table.shape[1]
    hi = jax.lax.Precision.HIGHEST

    ut, hu = _transpose_pack(user_table.T)
    it, hi_ = _transpose_pack(item_table.T)
    o1, o2, o3, o4, o5 = du, 2 * du, 2 * du + ds, 2 * du + 2 * ds, 2 * du + 3 * ds
    gc_tbl = jnp.dot(category_table, W1[o2:o3], precision=hi)
    gco_tbl = jnp.dot(color_table, W1[o3:o4], precision=hi)
    gp_tbl = jnp.dot(price_range_table, W1[o4:o5], precision=hi)

    upidx = jnp.where(user_id < hu, user_id, user_id - hu)
    ipidx = jnp.where(item_id < hi_, item_id, item_id - hi_)

    gather = _make_gather(batch)
    upk, ipk, gc, gco, gp = gather(
        ut, it, gc_tbl, gco_tbl, gp_tbl,
        upidx.reshape(-1, _CHUNK), ipidx.reshape(-1, _CHUNK),
        category.reshape(-1, _CHUNK), color.reshape(-1, _CHUNK),
        price_range.reshape(-1, _CHUNK))

    sf = jnp.stack([num_colors, price_numeric,
                    (user_id >= hu).astype(jnp.float32),
                    (item_id >= hi_).astype(jnp.float32)], axis=1)
    w1s = jnp.concatenate([W1[o5:], jnp.zeros((2, W1.shape[1]), jnp.float32)],
                         axis=0)
    out = _mlp(upk, ipk, gc, gco, gp, sf,
               W1[:o1], W1[o1:o2], w1s,
               b1.reshape(1, -1), W2, b2.reshape(1, -1), W3, b3.reshape(1, -1),
               W4, b4.reshape(1, 1))
    return out[:, 0]


# speedup vs baseline: 1.9253x; 1.2524x over previous
import functools

import jax
import jax.numpy as jnp
from jax import lax
from jax.experimental import pallas as pl
from jax.experimental.pallas import tpu as pltpu
from jax.experimental.pallas import tpu_sc as plsc

_NC = 2
_NS = 16
_NW = _NC * _NS
_CHUNK = 128


def _sc_mesh():
    return plsc.VectorSubcoreMesh(core_axis_name="c", subcore_axis_name="s")


def _gather_body(bpw, u_tbl, i_tbl, c_tbl, co_tbl, p_tbl,
                 uid, iid, cid, coid, pid,
                 u_out, i_out, c_out, co_out, p_out,
                 uix, iix, cix, coix, pix,
                 buf0, buf1, sem0, sem1, osem0, osem1):
    idx_rows = bpw // _CHUNK
    wid = lax.axis_index("s") * _NC + lax.axis_index("c")
    base = wid * bpw
    irow = wid * idx_rows
    fields = [(uid, uix, u_tbl, u_out),
              (iid, iix, i_tbl, i_out),
              (cid, cix, c_tbl, c_out),
              (coid, coix, co_tbl, co_out),
              (pid, pix, p_tbl, p_out)]
    for ids, ixv, _, _ in fields:
        pltpu.sync_copy(ids.at[pl.ds(irow, idx_rows)], ixv)

    bufs = (buf0, buf1)
    gsems = (sem0, sem1)
    osems = (osem0, osem1)
    chunks = []
    for _, ixv, tbl, outr in fields:
        for j in range(idx_rows):
            chunks.append((ixv, j, tbl, outr, base + j * _CHUNK))
    n = len(chunks)
    gcps = [None, None]
    scps = [None, None]

    def fire(k):
        ixv, j, tbl, outr, _ = chunks[k]
        b = k % 2
        gcps[b] = pltpu.async_copy(tbl.at[ixv.at[j]], bufs[b], gsems[b])

    fire(0)
    for k in range(n):
        b = k % 2
        ixv, j, tbl, outr, off = chunks[k]
        if k + 1 < n:
            nb = (k + 1) % 2
            if scps[nb] is not None:
                scps[nb].wait()
                scps[nb] = None
            fire(k + 1)
        gcps[b].wait()
        scps[b] = pltpu.async_copy(bufs[b], outr.at[pl.ds(off, _CHUNK)],
                                   osems[b])
    for b in range(2):
        if scps[b] is not None:
            scps[b].wait()


@functools.lru_cache(maxsize=None)
def _make_gather(batch):
    bpw = batch // _NW
    idx_rows = bpw // _CHUNK
    f32 = jnp.float32
    out = jax.ShapeDtypeStruct((batch, 128), f32)
    return pl.kernel(
        functools.partial(_gather_body, bpw),
        out_type=[out] * 5,
        mesh=_sc_mesh(),
        scratch_types=[pltpu.VMEM((idx_rows, _CHUNK), jnp.int32)] * 5
                      + [pltpu.VMEM((_CHUNK, 128), f32)] * 2
                      + [pltpu.SemaphoreType.DMA] * 4,
        compiler_params=pltpu.CompilerParams(use_tc_tiling_on_sc=True, needs_layout_passes=False),
    )


def _tpack_body(xa, xb, out):
    ya = jnp.transpose(xa[...])
    yb = jnp.transpose(xb[...])
    out[...] = jnp.concatenate([ya, yb], axis=1)


def _transpose_pack(table_t, tile=8192):
    n = table_t.shape[1]
    grid = -(-(n // 2) // tile)
    h = grid * tile
    last = n // tile
    packed = pl.pallas_call(
        _tpack_body,
        grid=(grid,),
        in_specs=[pl.BlockSpec((64, tile), lambda i: (0, i)),
                  pl.BlockSpec((64, tile),
                               lambda i: (0, jnp.minimum(i + grid, last)))],
        out_specs=pl.BlockSpec((tile, 128), lambda i: (i, 0)),
        out_shape=jax.ShapeDtypeStruct((h, 128), jnp.float32),
    )(table_t, table_t)
    return packed, h


def _mlp_body(upk, ipk, gc, gco, gp, sf,
              w1u, w1i, w1s, b1, w2, b2, w3, b3, w4, b4, out):
    f32 = jnp.float32
    hi = jax.lax.Precision.HIGHEST
    pu = sf[:, 2:3]
    pi = sf[:, 3:4]
    ue = jnp.where(pu > 0.5, upk[:, 64:128], upk[:, 0:64])
    ie = jnp.where(pi > 0.5, ipk[:, 64:128], ipk[:, 0:64])
    h = jnp.dot(ue, w1u[...], precision=hi, preferred_element_type=f32)
    h = h + jnp.dot(ie, w1i[...], precision=hi, preferred_element_type=f32)
    h = h + jnp.dot(sf[...], w1s[...], precision=hi, preferred_element_type=f32)
    h = h + gc[...] + gco[...] + gp[...] + b1[...]
    h = jnp.maximum(h, 0.0)
    h = jnp.maximum(jnp.dot(h, w2[...], precision=hi, preferred_element_type=f32) + b2[...], 0.0)
    h = jnp.maximum(jnp.dot(h, w3[...], precision=hi, preferred_element_type=f32) + b3[...], 0.0)
    o = jnp.dot(h, w4[...], precision=hi, preferred_element_type=f32) + b4[...]
    out[...] = jax.nn.sigmoid(o)


def _mlp(upk, ipk, gc, gco, gp, sf, w1u, w1i, w1s, b1,
         w2, b2, w3, b3, w4, b4, tile=1024, interpret=False):
    batch = upk.shape[0]

    def row(d):
        return pl.BlockSpec((tile, d), lambda i: (i, 0))

    def const(a):
        return pl.BlockSpec(a.shape, lambda i: (0,) * a.ndim)

    return pl.pallas_call(
        _mlp_body,
        grid=(batch // tile,),
        in_specs=[row(128), row(128), row(128), row(128), row(128),
                  row(sf.shape[1]),
                  const(w1u), const(w1i), const(w1s), const(b1),
                  const(w2), const(b2), const(w3), const(b3),
                  const(w4), const(b4)],
        out_specs=pl.BlockSpec((tile, 1), lambda i: (i, 0)),
        out_shape=jax.ShapeDtypeStruct((batch, 1), jnp.float32),
        interpret=interpret,
    )(upk, ipk, gc, gco, gp, sf, w1u, w1i, w1s, b1, w2, b2, w3, b3, w4, b4)


def kernel(user_id, item_id, category, color, price_range, num_colors,
           price_numeric, user_table, item_table, category_table, color_table,
           price_range_table, W1, b1, W2, b2, W3, b3, W4, b4):
    batch = user_id.shape[0]
    du = user_table.shape[1]
    ds = category_table.shape[1]
    hi = jax.lax.Precision.HIGHEST

    ut, hu = _transpose_pack(user_table.T)
    it, hi_ = _transpose_pack(item_table.T)
    o1, o2, o3, o4, o5 = du, 2 * du, 2 * du + ds, 2 * du + 2 * ds, 2 * du + 3 * ds
    gc_tbl = jnp.dot(category_table, W1[o2:o3], precision=hi)
    gco_tbl = jnp.dot(color_table, W1[o3:o4], precision=hi)
    gp_tbl = jnp.dot(price_range_table, W1[o4:o5], precision=hi)

    upidx = jnp.where(user_id < hu, user_id, user_id - hu)
    ipidx = jnp.where(item_id < hi_, item_id, item_id - hi_)

    gather = _make_gather(batch)
    upk, ipk, gc, gco, gp = gather(
        ut, it, gc_tbl, gco_tbl, gp_tbl,
        upidx.reshape(-1, _CHUNK), ipidx.reshape(-1, _CHUNK),
        category.reshape(-1, _CHUNK), color.reshape(-1, _CHUNK),
        price_range.reshape(-1, _CHUNK))

    sf = jnp.stack([num_colors, price_numeric,
                    (user_id >= hu).astype(jnp.float32),
                    (item_id >= hi_).astype(jnp.float32)], axis=1)
    w1s = jnp.concatenate([W1[o5:], jnp.zeros((2, W1.shape[1]), jnp.float32)],
                         axis=0)
    out = _mlp(upk, ipk, gc, gco, gp, sf,
               W1[:o1], W1[o1:o2], w1s,
               b1.reshape(1, -1), W2, b2.reshape(1, -1), W3, b3.reshape(1, -1),
               W4, b4.reshape(1, 1))
    return out[:, 0]


# speedup vs baseline: 1.9770x; 1.0269x over previous
import functools

import jax
import jax.numpy as jnp
from jax import lax
from jax.experimental import pallas as pl
from jax.experimental.pallas import tpu as pltpu
from jax.experimental.pallas import tpu_sc as plsc

_NC = 2
_NS = 16
_NW = _NC * _NS
_CHUNK = 128


def _sc_mesh():
    return plsc.VectorSubcoreMesh(core_axis_name="c", subcore_axis_name="s")


def _gather_body(bpw, u_tbl, i_tbl, c_tbl, co_tbl, p_tbl,
                 uid, iid, cid, coid, pid,
                 u_out, i_out, c_out, co_out, p_out,
                 uix, iix, cix, coix, pix,
                 buf0, buf1, sem0, sem1, osem0, osem1):
    idx_rows = bpw // _CHUNK
    wid = lax.axis_index("s") * _NC + lax.axis_index("c")
    base = wid * bpw
    irow = wid * idx_rows
    fields = [(uid, uix, u_tbl, u_out),
              (iid, iix, i_tbl, i_out),
              (cid, cix, c_tbl, c_out),
              (coid, coix, co_tbl, co_out),
              (pid, pix, p_tbl, p_out)]
    for ids, ixv, _, _ in fields:
        pltpu.sync_copy(ids.at[pl.ds(irow, idx_rows)], ixv)

    bufs = (buf0, buf1)
    gsems = (sem0, sem1)
    osems = (osem0, osem1)
    chunks = []
    for _, ixv, tbl, outr in fields:
        for j in range(idx_rows):
            chunks.append((ixv, j, tbl, outr, base + j * _CHUNK))
    n = len(chunks)
    gcps = [None, None]
    scps = [None, None]

    def fire(k):
        ixv, j, tbl, outr, _ = chunks[k]
        b = k % 2
        gcps[b] = pltpu.async_copy(tbl.at[ixv.at[j]], bufs[b], gsems[b])

    fire(0)
    for k in range(n):
        b = k % 2
        ixv, j, tbl, outr, off = chunks[k]
        if k + 1 < n:
            nb = (k + 1) % 2
            if scps[nb] is not None:
                scps[nb].wait()
                scps[nb] = None
            fire(k + 1)
        gcps[b].wait()
        scps[b] = pltpu.async_copy(bufs[b], outr.at[pl.ds(off, _CHUNK)],
                                   osems[b])
    for b in range(2):
        if scps[b] is not None:
            scps[b].wait()


@functools.lru_cache(maxsize=None)
def _make_gather(batch):
    bpw = batch // _NW
    idx_rows = bpw // _CHUNK
    f32 = jnp.float32
    out = jax.ShapeDtypeStruct((batch, 128), f32)
    return pl.kernel(
        functools.partial(_gather_body, bpw),
        out_type=[out] * 5,
        mesh=_sc_mesh(),
        scratch_types=[pltpu.VMEM((idx_rows, _CHUNK), jnp.int32)] * 5
                      + [pltpu.VMEM((_CHUNK, 128), f32)] * 2
                      + [pltpu.SemaphoreType.DMA] * 4,
        compiler_params=pltpu.CompilerParams(use_tc_tiling_on_sc=True, needs_layout_passes=False),
    )


def _tpack_body(xa, xb, out):
    ya = jnp.transpose(xa[...])
    yb = jnp.transpose(xb[...])
    out[...] = jnp.concatenate([ya, yb], axis=1)


def _transpose_pack(table_t, tile=16384):
    n = table_t.shape[1]
    grid = -(-(n // 2) // tile)
    h = grid * tile
    last = n // tile
    packed = pl.pallas_call(
        _tpack_body,
        grid=(grid,),
        in_specs=[pl.BlockSpec((64, tile), lambda i: (0, i)),
                  pl.BlockSpec((64, tile),
                               lambda i: (0, jnp.minimum(i + grid, last)))],
        out_specs=pl.BlockSpec((tile, 128), lambda i: (i, 0)),
        out_shape=jax.ShapeDtypeStruct((h, 128), jnp.float32),
    )(table_t, table_t)
    return packed, h


def _mlp_body(upk, ipk, gc, gco, gp, sf,
              w1u, w1i, w1s, b1, w2, b2, w3, b3, w4, b4, out):
    f32 = jnp.float32
    hi = jax.lax.Precision.HIGHEST
    pu = sf[:, 2:3]
    pi = sf[:, 3:4]
    ue = jnp.where(pu > 0.5, upk[:, 64:128], upk[:, 0:64])
    ie = jnp.where(pi > 0.5, ipk[:, 64:128], ipk[:, 0:64])
    h = jnp.dot(ue, w1u[...], precision=hi, preferred_element_type=f32)
    h = h + jnp.dot(ie, w1i[...], precision=hi, preferred_element_type=f32)
    h = h + jnp.dot(sf[...], w1s[...], precision=hi, preferred_element_type=f32)
    h = h + gc[...] + gco[...] + gp[...] + b1[...]
    h = jnp.maximum(h, 0.0)
    h = jnp.maximum(jnp.dot(h, w2[...], precision=hi, preferred_element_type=f32) + b2[...], 0.0)
    h = jnp.maximum(jnp.dot(h, w3[...], precision=hi, preferred_element_type=f32) + b3[...], 0.0)
    o = jnp.dot(h, w4[...], precision=hi, preferred_element_type=f32) + b4[...]
    out[...] = jax.nn.sigmoid(o)


def _mlp(upk, ipk, gc, gco, gp, sf, w1u, w1i, w1s, b1,
         w2, b2, w3, b3, w4, b4, tile=1024, interpret=False):
    batch = upk.shape[0]

    def row(d):
        return pl.BlockSpec((tile, d), lambda i: (i, 0))

    def const(a):
        return pl.BlockSpec(a.shape, lambda i: (0,) * a.ndim)

    return pl.pallas_call(
        _mlp_body,
        grid=(batch // tile,),
        in_specs=[row(128), row(128), row(128), row(128), row(128),
                  row(sf.shape[1]),
                  const(w1u), const(w1i), const(w1s), const(b1),
                  const(w2), const(b2), const(w3), const(b3),
                  const(w4), const(b4)],
        out_specs=pl.BlockSpec((tile, 1), lambda i: (i, 0)),
        out_shape=jax.ShapeDtypeStruct((batch, 1), jnp.float32),
        interpret=interpret,
    )(upk, ipk, gc, gco, gp, sf, w1u, w1i, w1s, b1, w2, b2, w3, b3, w4, b4)


def kernel(user_id, item_id, category, color, price_range, num_colors,
           price_numeric, user_table, item_table, category_table, color_table,
           price_range_table, W1, b1, W2, b2, W3, b3, W4, b4):
    batch = user_id.shape[0]
    du = user_table.shape[1]
    ds = category_table.shape[1]
    hi = jax.lax.Precision.HIGHEST

    ut, hu = _transpose_pack(user_table.T)
    it, hi_ = _transpose_pack(item_table.T)
    o1, o2, o3, o4, o5 = du, 2 * du, 2 * du + ds, 2 * du + 2 * ds, 2 * du + 3 * ds
    gc_tbl = jnp.dot(category_table, W1[o2:o3], precision=hi)
    gco_tbl = jnp.dot(color_table, W1[o3:o4], precision=hi)
    gp_tbl = jnp.dot(price_range_table, W1[o4:o5], precision=hi)

    upidx = jnp.where(user_id < hu, user_id, user_id - hu)
    ipidx = jnp.where(item_id < hi_, item_id, item_id - hi_)

    gather = _make_gather(batch)
    upk, ipk, gc, gco, gp = gather(
        ut, it, gc_tbl, gco_tbl, gp_tbl,
        upidx.reshape(-1, _CHUNK), ipidx.reshape(-1, _CHUNK),
        category.reshape(-1, _CHUNK), color.reshape(-1, _CHUNK),
        price_range.reshape(-1, _CHUNK))

    sf = jnp.stack([num_colors, price_numeric,
                    (user_id >= hu).astype(jnp.float32),
                    (item_id >= hi_).astype(jnp.float32)], axis=1)
    w1s = jnp.concatenate([W1[o5:], jnp.zeros((2, W1.shape[1]), jnp.float32)],
                         axis=0)
    out = _mlp(upk, ipk, gc, gco, gp, sf,
               W1[:o1], W1[o1:o2], w1s,
               b1.reshape(1, -1), W2, b2.reshape(1, -1), W3, b3.reshape(1, -1),
               W4, b4.reshape(1, 1))
    return out[:, 0]
